# baseline (speedup 1.0000x reference)
import functools

import jax
import jax.numpy as jnp
from jax import lax
from jax.experimental import pallas as pl
from jax.experimental.pallas import tpu as pltpu
from jax.experimental.pallas import tpu_sc as plsc

_BATCH = 16384
_DIM = 32
_CHUNK = 128

_BUILT = {}


def _build():
    if "fn" in _BUILT:
        return _BUILT["fn"]

    info = plsc.get_sparse_core_info()
    nc, ns = info.num_cores, info.num_subcores
    nw = nc * ns
    b_per_w = _BATCH // nw
    n_chunks = b_per_w // _CHUNK

    mesh = plsc.VectorSubcoreMesh(core_axis_name="c", subcore_axis_name="s")

    @functools.partial(
        pl.kernel,
        mesh=mesh,
        out_type=jax.ShapeDtypeStruct((_BATCH, _DIM), jnp.float32),
        scratch_types=[
            pltpu.VMEM((n_chunks, _CHUNK), jnp.int32),
            pltpu.VMEM((b_per_w, _DIM), jnp.float32),
            pltpu.SemaphoreType.DMA,
        ],
        compiler_params=pltpu.CompilerParams(use_tc_tiling_on_sc=False),
    )
    def gather_kernel(idx_hbm, table_hbm, out_hbm, idx_v, rows_v, sem):
        wid = lax.axis_index("s") * nc + lax.axis_index("c")
        base = wid * b_per_w
        pltpu.sync_copy(idx_hbm.at[wid], idx_v)
        copies = []
        for j in range(n_chunks):
            copies.append(
                pltpu.async_copy(
                    table_hbm.at[idx_v.at[j]],
                    rows_v.at[pl.ds(j * _CHUNK, _CHUNK)],
                    sem,
                )
            )
        for c in copies:
            c.wait()
        pltpu.sync_copy(rows_v, out_hbm.at[pl.ds(base, b_per_w)])

    def run(input_indices, embedding_matrix):
        idx3 = input_indices.reshape(nw, n_chunks, _CHUNK)
        return gather_kernel(idx3, embedding_matrix)

    _BUILT["fn"] = run
    return run


def kernel(input_indices, embedding_matrix):
    return _build()(input_indices, embedding_matrix)


# speedup vs baseline: 3.5729x; 3.5729x over previous
import functools

import jax
import jax.numpy as jnp
from jax import lax
from jax.experimental import pallas as pl
from jax.experimental.pallas import tpu as pltpu
from jax.experimental.pallas import tpu_sc as plsc

_BATCH = 16384
_DIM = 32
_LANES = 128
_K = 16

_BUILT = {}


def _build():
    if "fn" in _BUILT:
        return _BUILT["fn"]

    info = plsc.get_sparse_core_info()
    nc, ns = info.num_cores, info.num_subcores
    nw = nc * ns
    b_per_w = _BATCH // nw
    n_cohorts = b_per_w // _K

    mesh = plsc.VectorSubcoreMesh(core_axis_name="c", subcore_axis_name="s")

    @functools.partial(
        pl.kernel,
        mesh=mesh,
        out_type=jax.ShapeDtypeStruct((_DIM, _BATCH), jnp.float32),
        scratch_types=[
            pltpu.VMEM((b_per_w,), jnp.int32),
            pltpu.VMEM((_K, 4, 8, _LANES), jnp.float32),
            pltpu.VMEM((_DIM, b_per_w), jnp.float32),
            pltpu.SemaphoreType.DMA,
        ],
        compiler_params=pltpu.CompilerParams(
            use_tc_tiling_on_sc=True, needs_layout_passes=False
        ),
    )
    def gather_kernel(idx_hbm, tab4_hbm, out_t_hbm, idx_v, blocks_v, outc_v, sem):
        wid = lax.axis_index("s") * nc + lax.axis_index("c")
        base = pl.multiple_of(wid * b_per_w, b_per_w)
        pltpu.sync_copy(idx_hbm.at[pl.ds(base, b_per_w)], idx_v)

        d16 = lax.iota(jnp.int32, 16)
        r_lo, s_lo = d16 >> 3, d16 & 7
        r_hi, s_hi = (d16 + 16) >> 3, d16 & 7

        def per_cohort(g, _):
            vec = idx_v[pl.ds(g * _K, _K)]
            copies = []
            for k in range(_K):
                s = vec[k]
                col = pl.multiple_of((s >> 7) * _LANES, _LANES)
                copies.append(
                    pltpu.async_copy(
                        tab4_hbm.at[:, :, pl.ds(col, _LANES)],
                        blocks_v.at[k],
                        sem,
                    )
                )
            for c in copies:
                c.wait()
            lanes = vec & 127
            for k in range(_K):
                ksp = jnp.full((16,), k, jnp.int32)
                lsp = jnp.full((16,), lanes[k], jnp.int32)
                jsp = jnp.full((16,), g * _K + k, jnp.int32)
                lo = plsc.load_gather(blocks_v, [ksp, r_lo, s_lo, lsp])
                hi = plsc.load_gather(blocks_v, [ksp, r_hi, s_hi, lsp])
                plsc.store_scatter(outc_v, [d16, jsp], lo)
                plsc.store_scatter(outc_v, [d16 + 16, jsp], hi)
            return _

        lax.fori_loop(0, n_cohorts, per_cohort, None)
        pltpu.sync_copy(outc_v, out_t_hbm.at[:, pl.ds(base, b_per_w)])

    def run(input_indices, embedding_matrix):
        tab4 = embedding_matrix.T.reshape(4, 8, embedding_matrix.shape[0])
        out_t = gather_kernel(input_indices, tab4)
        return out_t.T

    _BUILT["fn"] = run
    return run


def kernel(input_indices, embedding_matrix):
    return _build()(input_indices, embedding_matrix)


# speedup vs baseline: 3.8204x; 1.0693x over previous
import functools

import jax
import jax.numpy as jnp
from jax import lax
from jax.experimental import pallas as pl
from jax.experimental.pallas import tpu as pltpu
from jax.experimental.pallas import tpu_sc as plsc

_BATCH = 16384
_DIM = 32
_LANES = 128
_K = 8

_BUILT = {}


def _build():
    if "fn" in _BUILT:
        return _BUILT["fn"]

    info = plsc.get_sparse_core_info()
    nc, ns = info.num_cores, info.num_subcores
    nw = nc * ns
    b_per_w = _BATCH // nw
    n_coh = b_per_w // _K

    mesh = plsc.VectorSubcoreMesh(core_axis_name="c", subcore_axis_name="s")

    @functools.partial(
        pl.kernel,
        mesh=mesh,
        out_type=jax.ShapeDtypeStruct((_DIM, _BATCH), jnp.float32),
        scratch_types=[
            pltpu.VMEM((b_per_w + 16, ), jnp.int32),
            pltpu.VMEM((2, _K, 4, 8, _LANES), jnp.float32),
            pltpu.VMEM((_DIM, b_per_w), jnp.float32),
            pltpu.SemaphoreType.DMA,
            pltpu.SemaphoreType.DMA,
        ],
        compiler_params=pltpu.CompilerParams(
            use_tc_tiling_on_sc=True, needs_layout_passes=False
        ),
    )
    def gather_kernel(
        idx_hbm, tab4_hbm, out_t_hbm, idx_v, blocks_v, outc_v, sem0, sem1
    ):
        wid = lax.axis_index("s") * nc + lax.axis_index("c")
        base = pl.multiple_of(wid * b_per_w, b_per_w)
        pltpu.sync_copy(idx_hbm.at[pl.ds(base, b_per_w)], idx_v.at[pl.ds(0, b_per_w)])

        sems = (sem0, sem1)
        d16 = lax.iota(jnp.int32, 16)
        r_lo, s_lo = d16 >> 3, d16 & 7
        r_hi = (d16 + 16) >> 3

        def fire(g, par):
            vec = idx_v[pl.ds(g * _K, 16)]
            for k in range(_K):
                s = vec[k]
                col = pl.multiple_of((s >> 7) * _LANES, _LANES)
                pltpu.async_copy(
                    tab4_hbm.at[:, :, pl.ds(col, _LANES)],
                    blocks_v.at[par, k],
                    sems[par],
                )

        def drain(par):
            for k in range(_K):
                pltpu.make_async_copy(
                    tab4_hbm.at[:, :, pl.ds(0, _LANES)],
                    blocks_v.at[par, k],
                    sems[par],
                ).wait()

        def process(g, par):
            vec = idx_v[pl.ds(g * _K, 16)]
            lanes = vec & 127
            psp = jnp.full((16,), par, jnp.int32)
            for k in range(_K):
                ksp = jnp.full((16,), k, jnp.int32)
                lsp = jnp.full((16,), lanes[k], jnp.int32)
                jsp = jnp.full((16,), g * _K + k, jnp.int32)
                lo = plsc.load_gather(blocks_v, [psp, ksp, r_lo, s_lo, lsp])
                hi = plsc.load_gather(blocks_v, [psp, ksp, r_hi, s_lo, lsp])
                plsc.store_scatter(outc_v, [d16, jsp], lo)
                plsc.store_scatter(outc_v, [d16 + 16, jsp], hi)

        def step(g, cur, nxt):
            @pl.when(g + 1 < n_coh)
            def _():
                fire(g + 1, nxt)

            drain(cur)
            process(g, cur)

        fire(0, 0)

        def body(g, _):
            @pl.when((g & 1) == 0)
            def _():
                step(g, 0, 1)

            @pl.when((g & 1) == 1)
            def _():
                step(g, 1, 0)

            return _

        lax.fori_loop(0, n_coh, body, None)
        pltpu.sync_copy(outc_v, out_t_hbm.at[:, pl.ds(base, b_per_w)])

    def run(input_indices, embedding_matrix):
        tab4 = embedding_matrix.T.reshape(4, 8, embedding_matrix.shape[0])
        out_t = gather_kernel(input_indices, tab4)
        return out_t.T

    _BUILT["fn"] = run
    return run


def kernel(input_indices, embedding_matrix):
    return _build()(input_indices, embedding_matrix)


# speedup vs baseline: 4.1517x; 1.0867x over previous
import functools

import jax
import jax.numpy as jnp
from jax import lax
from jax.experimental import pallas as pl
from jax.experimental.pallas import tpu as pltpu
from jax.experimental.pallas import tpu_sc as plsc

_BATCH = 16384
_DIM = 32
_LANES = 128
_K = 8

_BUILT = {}


def _build():
    if "fn" in _BUILT:
        return _BUILT["fn"]

    info = plsc.get_sparse_core_info()
    nc, ns = info.num_cores, info.num_subcores
    nw = nc * ns
    b_per_w = _BATCH // nw
    n_coh = b_per_w // _K

    mesh = plsc.VectorSubcoreMesh(core_axis_name="c", subcore_axis_name="s")

    @functools.partial(
        pl.kernel,
        mesh=mesh,
        out_type=jax.ShapeDtypeStruct((_DIM, _BATCH), jnp.float32),
        scratch_types=[
            pltpu.VMEM((b_per_w + 16, ), jnp.int32),
            pltpu.VMEM((3, _K, 4, 8, _LANES), jnp.float32),
            pltpu.VMEM((_DIM, b_per_w), jnp.float32),
            pltpu.SemaphoreType.DMA,
            pltpu.SemaphoreType.DMA,
            pltpu.SemaphoreType.DMA,
        ],
        compiler_params=pltpu.CompilerParams(
            use_tc_tiling_on_sc=True, needs_layout_passes=False
        ),
    )
    def gather_kernel(
        idx_hbm, tab4_hbm, out_t_hbm, idx_v, blocks_v, outc_v, sem0, sem1, sem2
    ):
        wid = lax.axis_index("s") * nc + lax.axis_index("c")
        base = pl.multiple_of(wid * b_per_w, b_per_w)
        pltpu.sync_copy(idx_hbm.at[pl.ds(base, b_per_w)], idx_v.at[pl.ds(0, b_per_w)])

        sems = (sem0, sem1, sem2)
        d16 = lax.iota(jnp.int32, 16)
        r_lo, s_lo = d16 >> 3, d16 & 7
        r_hi = (d16 + 16) >> 3

        def fire(g, par):
            vec = idx_v[pl.ds(g * _K, 16)]
            for k in range(_K):
                s = vec[k]
                col = pl.multiple_of((s >> 7) * _LANES, _LANES)
                pltpu.async_copy(
                    tab4_hbm.at[:, :, pl.ds(col, _LANES)],
                    blocks_v.at[par, k],
                    sems[par],
                )

        def drain(par):
            for k in range(_K):
                pltpu.make_async_copy(
                    tab4_hbm.at[:, :, pl.ds(0, _LANES)],
                    blocks_v.at[par, k],
                    sems[par],
                ).wait()

        def process(g, par):
            vec = idx_v[pl.ds(g * _K, 16)]
            lanes = vec & 127
            psp = jnp.full((16,), par, jnp.int32)
            for k in range(_K):
                ksp = jnp.full((16,), k, jnp.int32)
                lsp = jnp.full((16,), lanes[k], jnp.int32)
                jsp = jnp.full((16,), g * _K + k, jnp.int32)
                lo = plsc.load_gather(blocks_v, [psp, ksp, r_lo, s_lo, lsp])
                hi = plsc.load_gather(blocks_v, [psp, ksp, r_hi, s_lo, lsp])
                plsc.store_scatter(outc_v, [d16, jsp], lo)
                plsc.store_scatter(outc_v, [d16 + 16, jsp], hi)

        def step(g, cur):
            @pl.when(g + 2 < n_coh)
            def _():
                fire(g + 2, (cur + 2) % 3)

            drain(cur)
            process(g, cur)

        fire(0, 0)
        fire(1, 1)

        def body(g, _):
            par = lax.rem(g, 3)
            for p in range(3):
                @pl.when(par == p)
                def _(p=p):
                    step(g, p)

            return _

        lax.fori_loop(0, n_coh, body, None)
        pltpu.sync_copy(outc_v, out_t_hbm.at[:, pl.ds(base, b_per_w)])

    def run(input_indices, embedding_matrix):
        tab4 = embedding_matrix.T.reshape(4, 8, embedding_matrix.shape[0])
        out_t = gather_kernel(input_indices, tab4)
        return out_t.T

    _BUILT["fn"] = run
    return run


def kernel(input_indices, embedding_matrix):
    return _build()(input_indices, embedding_matrix)
